# baseline (speedup 1.0000x reference)
import jax
import jax.numpy as jnp
from jax import lax
from jax.experimental import pallas as pl
from jax.experimental.pallas import tpu as pltpu
from jax.experimental.pallas import tpu_sc as plsc

N = 10000
E = 320000
D = 128
NC = 2
NS = 16
NW = NC * NS
CH = 128
EPC = 80
NCHP = NW * EPC
EP = NCHP * CH
HEPC = EPC // 2
NA = 10240
NAD = 11000

_mesh = plsc.VectorSubcoreMesh(
    core_axis_name="c", subcore_axis_name="s", num_cores=NC, num_subcores=NS)

_PREC = lax.Precision.HIGHEST


def _zero_vmem_2d(ref, rows):
    def body(i, carry):
        r = i // (D // 16)
        col = (i % (D // 16)) * 16
        ref[r, pl.ds(col, 16)] = jnp.zeros((16,), jnp.float32)
        return carry
    lax.fori_loop(0, rows * (D // 16), body, 0)


def _sc_degrees_body(srcd_hbm, dstd_hbm, out_hbm, sblk, dblk, ones, zbuf,
                     dsp_out, dsp_in, sem_a, sem_b):
    c = lax.axis_index("c")
    s = lax.axis_index("s")
    w = s * NC + c

    def fill_ones(i, carry):
        ones[pl.ds(i * 16, 16)] = jnp.full((16,), 1.0, jnp.float32)
        return carry
    lax.fori_loop(0, CH // 16, fill_ones, 0)

    def fill_zeros(i, carry):
        zbuf[pl.ds(i * 16, 16)] = jnp.zeros((16,), jnp.float32)
        return carry
    lax.fori_loop(0, 64, fill_zeros, 0)

    pltpu.sync_copy(srcd_hbm.at[pl.ds(w * EPC, EPC)], sblk)
    pltpu.sync_copy(dstd_hbm.at[pl.ds(w * EPC, EPC)], dblk)

    @pl.when(s < NAD // 1000)
    def _():
        pltpu.sync_copy(zbuf.at[pl.ds(0, 1000)], dsp_out.at[pl.ds(s * 1000, 1000)])
        pltpu.sync_copy(zbuf.at[pl.ds(0, 1000)], dsp_in.at[pl.ds(s * 1000, 1000)])
    plsc.subcore_barrier()

    def group(t, carry):
        ds_ = []
        for b in range(8):
            j = t * 8 + b
            ds_.append(pltpu.async_copy(ones, dsp_out.at[sblk.at[j]], sem_a,
                                        add=True))
            ds_.append(pltpu.async_copy(ones, dsp_in.at[dblk.at[j]], sem_b,
                                        add=True))
        for d in ds_:
            d.wait()
        return carry
    lax.fori_loop(0, EPC // 8, group, 0)
    plsc.subcore_barrier()

    @pl.when(s < 10)
    def _():
        pltpu.sync_copy(dsp_out.at[pl.ds(s * 1000, 1000)], zbuf.at[pl.ds(0, 1000)])
        pltpu.sync_copy(zbuf.at[pl.ds(0, 1000)],
                        out_hbm.at[pl.ds(c * 2 * N + s * 1000, 1000)])
        pltpu.sync_copy(dsp_in.at[pl.ds(s * 1000, 1000)], zbuf.at[pl.ds(0, 1000)])
        pltpu.sync_copy(zbuf.at[pl.ds(0, 1000)],
                        out_hbm.at[pl.ds(c * 2 * N + N + s * 1000, 1000)])


def _make_degrees(interpret=False):
    return pl.kernel(
        _sc_degrees_body,
        out_type=jax.ShapeDtypeStruct((NC * 2 * N,), jnp.float32),
        mesh=_mesh,
        scratch_types=[
            pltpu.VMEM((EPC, CH), jnp.int32),
            pltpu.VMEM((EPC, CH), jnp.int32),
            pltpu.VMEM((CH,), jnp.float32),
            pltpu.VMEM((1024,), jnp.float32),
            pltpu.VMEM_SHARED((NAD,), jnp.float32),
            pltpu.VMEM_SHARED((NAD,), jnp.float32),
            pltpu.SemaphoreType.DMA,
            pltpu.SemaphoreType.DMA,
        ],
        interpret=interpret,
    )


_sc_degrees = _make_degrees()


def _sc_scatter_body(g_hbm, srcg_hbm, dstg_hbm, out_hbm, sblk, dblk,
                     stage0, stage1, accum, sem0, sem1, ssem0, ssem1):
    c = lax.axis_index("c")
    s = lax.axis_index("s")
    w = s * NC + c

    _zero_vmem_2d(stage0, CH)
    for j in range(5):
        cidx = s + j * NS
        pltpu.sync_copy(stage0, accum.at[pl.ds(cidx * CH, CH)])
    plsc.subcore_barrier()

    def wait_gather(stage, gsem):
        pltpu.make_async_copy(g_hbm.at[pl.ds(0, CH)], stage, gsem).wait()

    def wait_scatter(stage, dref, ssem):
        pltpu.make_async_copy(stage, accum.at[dref], ssem).wait()

    for h in range(2):
        base = w * EPC + h * HEPC
        pltpu.sync_copy(srcg_hbm.at[pl.ds(base, HEPC)], sblk)
        pltpu.sync_copy(dstg_hbm.at[pl.ds(base, HEPC)], dblk)
        pltpu.async_copy(g_hbm.at[sblk.at[0]], stage0, sem0)
        pltpu.async_copy(g_hbm.at[sblk.at[1]], stage1, sem1)

        def body(t, carry):
            e = 2 * t
            wait_gather(stage0, sem0)
            pltpu.async_copy(stage0, accum.at[dblk.at[e]], ssem0, add=True)
            wait_gather(stage1, sem1)
            pltpu.async_copy(stage1, accum.at[dblk.at[e + 1]], ssem1, add=True)

            @pl.when(t < HEPC // 2 - 1)
            def _():
                wait_scatter(stage0, dblk.at[e], ssem0)
                pltpu.async_copy(g_hbm.at[sblk.at[e + 2]], stage0, sem0)
                wait_scatter(stage1, dblk.at[e + 1], ssem1)
                pltpu.async_copy(g_hbm.at[sblk.at[e + 3]], stage1, sem1)
            return carry
        lax.fori_loop(0, HEPC // 2, body, 0)
        wait_scatter(stage0, dblk.at[HEPC - 2], ssem0)
        wait_scatter(stage1, dblk.at[HEPC - 1], ssem1)
    plsc.subcore_barrier()

    for j in range(8):
        cidx = s + j * NS
        @pl.when(cidx < 125)
        def _():
            rr = cidx * 80
            pltpu.sync_copy(accum.at[pl.ds(rr, 80)], stage0.at[pl.ds(0, 80)])
            pltpu.sync_copy(stage0.at[pl.ds(0, 80)], out_hbm.at[c, pl.ds(rr, 80)])


def _make_scatter(interpret=False):
    return pl.kernel(
        _sc_scatter_body,
        out_type=jax.ShapeDtypeStruct((NC, N, D), jnp.float32),
        mesh=_mesh,
        scratch_types=[
            pltpu.VMEM((HEPC, CH), jnp.int32),
            pltpu.VMEM((HEPC, CH), jnp.int32),
            pltpu.VMEM((CH, D), jnp.float32),
            pltpu.VMEM((CH, D), jnp.float32),
            pltpu.VMEM_SHARED((NA, D), jnp.float32),
            pltpu.SemaphoreType.DMA,
            pltpu.SemaphoreType.DMA,
            pltpu.SemaphoreType.DMA,
            pltpu.SemaphoreType.DMA,
        ],
        interpret=interpret,
    )


_sc_scatter = _make_scatter()


_BLK = 1000
_GRID = N // _BLK


def _stage1_body(x_ref, do0, do1, di0, di1, w1_ref, g_ref, no_ref, ni_ref):
    deg_o = do0[...] + do1[...]
    deg_i = di0[...] + di1[...]
    n_out = lax.rsqrt(jnp.maximum(deg_o, 1.0))
    n_in = lax.rsqrt(jnp.maximum(deg_i, 1.0))
    h = x_ref[...] * n_out
    g_ref[...] = jnp.dot(h, w1_ref[...], preferred_element_type=jnp.float32,
                         precision=_PREC)
    no_ref[...] = n_out
    ni_ref[...] = n_in


_stage1 = pl.pallas_call(
    _stage1_body,
    grid=(_GRID,),
    in_specs=[
        pl.BlockSpec((_BLK, D), lambda i: (i, 0)),
        pl.BlockSpec((_BLK, 1), lambda i: (i, 0)),
        pl.BlockSpec((_BLK, 1), lambda i: (i, 0)),
        pl.BlockSpec((_BLK, 1), lambda i: (i, 0)),
        pl.BlockSpec((_BLK, 1), lambda i: (i, 0)),
        pl.BlockSpec((D, D), lambda i: (0, 0)),
    ],
    out_specs=[
        pl.BlockSpec((_BLK, D), lambda i: (i, 0)),
        pl.BlockSpec((_BLK, 1), lambda i: (i, 0)),
        pl.BlockSpec((_BLK, 1), lambda i: (i, 0)),
    ],
    out_shape=[
        jax.ShapeDtypeStruct((N, D), jnp.float32),
        jax.ShapeDtypeStruct((N, 1), jnp.float32),
        jax.ShapeDtypeStruct((N, 1), jnp.float32),
    ],
)


def _stage2_body(aggp_ref, ni_ref, no_ref, b_ref, w_ref, g_ref):
    agg = aggp_ref[0] + aggp_ref[1]
    h = jnp.maximum(agg * ni_ref[...] + b_ref[...], 0.0)
    g_ref[...] = jnp.dot(h * no_ref[...], w_ref[...],
                         preferred_element_type=jnp.float32, precision=_PREC)


_stage2 = pl.pallas_call(
    _stage2_body,
    grid=(_GRID,),
    in_specs=[
        pl.BlockSpec((NC, _BLK, D), lambda i: (0, i, 0)),
        pl.BlockSpec((_BLK, 1), lambda i: (i, 0)),
        pl.BlockSpec((_BLK, 1), lambda i: (i, 0)),
        pl.BlockSpec((1, D), lambda i: (0, 0)),
        pl.BlockSpec((D, D), lambda i: (0, 0)),
    ],
    out_specs=pl.BlockSpec((_BLK, D), lambda i: (i, 0)),
    out_shape=jax.ShapeDtypeStruct((N, D), jnp.float32),
)


def _stage3_body(aggp_ref, ni_ref, b_ref, wl_ref, bl_ref, out_ref, acc_ref):
    i = pl.program_id(0)

    @pl.when(i == 0)
    def _():
        acc_ref[...] = jnp.zeros_like(acc_ref)

    agg = aggp_ref[0] + aggp_ref[1]
    h = jnp.maximum(agg * ni_ref[...] + b_ref[...], 0.0)
    acc_ref[...] += jnp.sum(h, axis=0, keepdims=True)

    @pl.when(i == _GRID - 1)
    def _():
        pooled = acc_ref[...] / jnp.float32(N)
        out_ref[...] = jnp.dot(pooled, wl_ref[...],
                               preferred_element_type=jnp.float32,
                               precision=_PREC) + bl_ref[...]


_stage3 = pl.pallas_call(
    _stage3_body,
    grid=(_GRID,),
    in_specs=[
        pl.BlockSpec((NC, _BLK, D), lambda i: (0, i, 0)),
        pl.BlockSpec((_BLK, 1), lambda i: (i, 0)),
        pl.BlockSpec((1, D), lambda i: (0, 0)),
        pl.BlockSpec((D, 1), lambda i: (0, 0)),
        pl.BlockSpec((1, 1), lambda i: (0, 0)),
    ],
    out_specs=pl.BlockSpec((1, 1), lambda i: (0, 0)),
    out_shape=jax.ShapeDtypeStruct((1, 1), jnp.float32),
    scratch_shapes=[pltpu.VMEM((1, D), jnp.float32)],
)


def kernel(x, edge_index, W1, b1, W2, b2, Wl, bl):
    src = edge_index[0].astype(jnp.int32)
    dst = edge_index[1].astype(jnp.int32)

    ar = jnp.arange(EP - E, dtype=jnp.int32)
    pad_dst = N + ar % (NA - N)
    pad_deg = N + ar % (NAD - N)
    pad_gat = (ar * 41) % N
    dstp = jnp.concatenate([dst, pad_dst]).reshape(NCHP, CH)
    srcd = jnp.concatenate([src, pad_deg]).reshape(NCHP, CH)
    srcg = jnp.concatenate([src, pad_gat]).reshape(NCHP, CH)

    degp = _sc_degrees(srcd, dstp).reshape(NC, 2, N)
    do0 = degp[0, 0].reshape(N, 1)
    do1 = degp[1, 0].reshape(N, 1)
    di0 = degp[0, 1].reshape(N, 1)
    di1 = degp[1, 1].reshape(N, 1)

    g1, n_out, n_in = _stage1(x, do0, do1, di0, di1, W1)
    agg1p = _sc_scatter(g1, srcg, dstp)
    g2 = _stage2(agg1p, n_in, n_out, b1.reshape(1, D), W2)
    agg2p = _sc_scatter(g2, srcg, dstp)
    out = _stage3(agg2p, n_in, b2.reshape(1, D), Wl, bl.reshape(1, 1))
    return out


# speedup vs baseline: 1.2323x; 1.2323x over previous
import jax
import jax.numpy as jnp
from jax import lax
from jax.experimental import pallas as pl
from jax.experimental.pallas import tpu as pltpu
from jax.experimental.pallas import tpu_sc as plsc

N = 10000
E = 320000
D = 128
NC = 2
NS = 16
NW = NC * NS
CH = 128
EPC = 80
NCHP = NW * EPC
EP = NCHP * CH
HEPC = EPC // 2
NA = 10240
NAD = 11000

_mesh = plsc.VectorSubcoreMesh(
    core_axis_name="c", subcore_axis_name="s", num_cores=NC, num_subcores=NS)

_PREC = lax.Precision.HIGHEST


def _zero_vmem_2d(ref, rows):
    def body(i, carry):
        r = i // (D // 16)
        col = (i % (D // 16)) * 16
        ref[r, pl.ds(col, 16)] = jnp.zeros((16,), jnp.float32)
        return carry
    lax.fori_loop(0, rows * (D // 16), body, 0)


def _sc_degrees_body(srcd_hbm, dstd_hbm, out_hbm, sblk, dblk, ones, zbuf,
                     dsp_out, dsp_in, sem_a, sem_b):
    c = lax.axis_index("c")
    s = lax.axis_index("s")
    w = s * NC + c

    def fill_ones(i, carry):
        ones[pl.ds(i * 16, 16)] = jnp.full((16,), 1.0, jnp.float32)
        return carry
    lax.fori_loop(0, CH // 16, fill_ones, 0)

    def fill_zeros(i, carry):
        zbuf[pl.ds(i * 16, 16)] = jnp.zeros((16,), jnp.float32)
        return carry
    lax.fori_loop(0, 64, fill_zeros, 0)

    pltpu.sync_copy(srcd_hbm.at[pl.ds(w * EPC, EPC)], sblk)
    pltpu.sync_copy(dstd_hbm.at[pl.ds(w * EPC, EPC)], dblk)

    @pl.when(s < NAD // 1000)
    def _():
        pltpu.sync_copy(zbuf.at[pl.ds(0, 1000)], dsp_out.at[pl.ds(s * 1000, 1000)])
        pltpu.sync_copy(zbuf.at[pl.ds(0, 1000)], dsp_in.at[pl.ds(s * 1000, 1000)])
    plsc.subcore_barrier()

    def group(t, carry):
        ds_ = []
        for b in range(8):
            j = t * 8 + b
            ds_.append(pltpu.async_copy(ones, dsp_out.at[sblk.at[j]], sem_a,
                                        add=True))
            ds_.append(pltpu.async_copy(ones, dsp_in.at[dblk.at[j]], sem_b,
                                        add=True))
        for d in ds_:
            d.wait()
        return carry
    lax.fori_loop(0, EPC // 8, group, 0)
    plsc.subcore_barrier()

    @pl.when(s < 10)
    def _():
        pltpu.sync_copy(dsp_out.at[pl.ds(s * 1000, 1000)], zbuf.at[pl.ds(0, 1000)])
        pltpu.sync_copy(zbuf.at[pl.ds(0, 1000)],
                        out_hbm.at[pl.ds(c * 2 * N + s * 1000, 1000)])
        pltpu.sync_copy(dsp_in.at[pl.ds(s * 1000, 1000)], zbuf.at[pl.ds(0, 1000)])
        pltpu.sync_copy(zbuf.at[pl.ds(0, 1000)],
                        out_hbm.at[pl.ds(c * 2 * N + N + s * 1000, 1000)])


def _make_degrees(interpret=False):
    return pl.kernel(
        _sc_degrees_body,
        out_type=jax.ShapeDtypeStruct((NC * 2 * N,), jnp.float32),
        mesh=_mesh,
        scratch_types=[
            pltpu.VMEM((EPC, CH), jnp.int32),
            pltpu.VMEM((EPC, CH), jnp.int32),
            pltpu.VMEM((CH,), jnp.float32),
            pltpu.VMEM((1024,), jnp.float32),
            pltpu.VMEM_SHARED((NAD,), jnp.float32),
            pltpu.VMEM_SHARED((NAD,), jnp.float32),
            pltpu.SemaphoreType.DMA,
            pltpu.SemaphoreType.DMA,
        ],
        interpret=interpret,
    )


_sc_degrees = _make_degrees()


def _sc_scatter_body(g_hbm, srcg_hbm, dstg_hbm, out_hbm, sblk, dblk,
                     stage0, stage1, accum, sem0, sem1):
    c = lax.axis_index("c")
    s = lax.axis_index("s")
    w = s * NC + c

    _zero_vmem_2d(stage0, CH)
    for j in range(5):
        cidx = s + j * NS
        pltpu.sync_copy(stage0, accum.at[pl.ds(cidx * CH, CH)])
    plsc.subcore_barrier()

    for h in range(2):
        base = w * EPC + h * HEPC
        pltpu.sync_copy(srcg_hbm.at[pl.ds(base, HEPC)], sblk)
        pltpu.sync_copy(dstg_hbm.at[pl.ds(base, HEPC)], dblk)
        pltpu.async_copy(g_hbm.at[sblk.at[0]], stage0, sem0)

        def body(t, carry):
            e = 2 * t
            d1 = pltpu.async_copy(g_hbm.at[sblk.at[e + 1]], stage1, sem1)
            pltpu.make_async_copy(g_hbm.at[pl.ds(0, CH)], stage0, sem0).wait()
            pltpu.sync_copy(stage0, accum.at[dblk.at[e]], add=True)

            @pl.when(t < HEPC // 2 - 1)
            def _():
                pltpu.async_copy(g_hbm.at[sblk.at[e + 2]], stage0, sem0)

            d1.wait()
            pltpu.sync_copy(stage1, accum.at[dblk.at[e + 1]], add=True)
            return carry
        lax.fori_loop(0, HEPC // 2, body, 0)
    plsc.subcore_barrier()

    for j in range(8):
        cidx = s + j * NS
        @pl.when(cidx < 125)
        def _():
            rr = cidx * 80
            pltpu.sync_copy(accum.at[pl.ds(rr, 80)], stage0.at[pl.ds(0, 80)])
            pltpu.sync_copy(stage0.at[pl.ds(0, 80)], out_hbm.at[c, pl.ds(rr, 80)])


def _make_scatter(interpret=False):
    return pl.kernel(
        _sc_scatter_body,
        out_type=jax.ShapeDtypeStruct((NC, N, D), jnp.float32),
        mesh=_mesh,
        scratch_types=[
            pltpu.VMEM((HEPC, CH), jnp.int32),
            pltpu.VMEM((HEPC, CH), jnp.int32),
            pltpu.VMEM((CH, D), jnp.float32),
            pltpu.VMEM((CH, D), jnp.float32),
            pltpu.VMEM_SHARED((NA, D), jnp.float32),
            pltpu.SemaphoreType.DMA,
            pltpu.SemaphoreType.DMA,
        ],
        interpret=interpret,
    )


_sc_scatter = _make_scatter()


_BLK = 1000
_GRID = N // _BLK


def _mm1_body(x_ref, w1_ref, p_ref):
    p_ref[...] = jnp.dot(x_ref[...], w1_ref[...],
                         preferred_element_type=jnp.float32, precision=_PREC)


_mm1 = pl.pallas_call(
    _mm1_body,
    grid=(_GRID,),
    in_specs=[
        pl.BlockSpec((_BLK, D), lambda i: (i, 0)),
        pl.BlockSpec((D, D), lambda i: (0, 0)),
    ],
    out_specs=pl.BlockSpec((_BLK, D), lambda i: (i, 0)),
    out_shape=jax.ShapeDtypeStruct((N, D), jnp.float32),
)


def _stage1_body(p_ref, do0, do1, di0, di1, g_ref, no_ref, ni_ref):
    deg_o = do0[...] + do1[...]
    deg_i = di0[...] + di1[...]
    n_out = lax.rsqrt(jnp.maximum(deg_o, 1.0))
    n_in = lax.rsqrt(jnp.maximum(deg_i, 1.0))
    g_ref[...] = p_ref[...] * n_out
    no_ref[...] = n_out
    ni_ref[...] = n_in


_stage1 = pl.pallas_call(
    _stage1_body,
    grid=(_GRID,),
    in_specs=[
        pl.BlockSpec((_BLK, D), lambda i: (i, 0)),
        pl.BlockSpec((_BLK, 1), lambda i: (i, 0)),
        pl.BlockSpec((_BLK, 1), lambda i: (i, 0)),
        pl.BlockSpec((_BLK, 1), lambda i: (i, 0)),
        pl.BlockSpec((_BLK, 1), lambda i: (i, 0)),
    ],
    out_specs=[
        pl.BlockSpec((_BLK, D), lambda i: (i, 0)),
        pl.BlockSpec((_BLK, 1), lambda i: (i, 0)),
        pl.BlockSpec((_BLK, 1), lambda i: (i, 0)),
    ],
    out_shape=[
        jax.ShapeDtypeStruct((N, D), jnp.float32),
        jax.ShapeDtypeStruct((N, 1), jnp.float32),
        jax.ShapeDtypeStruct((N, 1), jnp.float32),
    ],
)


def _stage2_body(aggp_ref, ni_ref, no_ref, b_ref, w_ref, g_ref):
    agg = aggp_ref[0] + aggp_ref[1]
    h = jnp.maximum(agg * ni_ref[...] + b_ref[...], 0.0)
    g_ref[...] = jnp.dot(h * no_ref[...], w_ref[...],
                         preferred_element_type=jnp.float32, precision=_PREC)


_stage2 = pl.pallas_call(
    _stage2_body,
    grid=(_GRID,),
    in_specs=[
        pl.BlockSpec((NC, _BLK, D), lambda i: (0, i, 0)),
        pl.BlockSpec((_BLK, 1), lambda i: (i, 0)),
        pl.BlockSpec((_BLK, 1), lambda i: (i, 0)),
        pl.BlockSpec((1, D), lambda i: (0, 0)),
        pl.BlockSpec((D, D), lambda i: (0, 0)),
    ],
    out_specs=pl.BlockSpec((_BLK, D), lambda i: (i, 0)),
    out_shape=jax.ShapeDtypeStruct((N, D), jnp.float32),
)


def _stage3_body(aggp_ref, ni_ref, b_ref, wl_ref, bl_ref, out_ref, acc_ref):
    i = pl.program_id(0)

    @pl.when(i == 0)
    def _():
        acc_ref[...] = jnp.zeros_like(acc_ref)

    agg = aggp_ref[0] + aggp_ref[1]
    h = jnp.maximum(agg * ni_ref[...] + b_ref[...], 0.0)
    acc_ref[...] += jnp.sum(h, axis=0, keepdims=True)

    @pl.when(i == _GRID - 1)
    def _():
        pooled = acc_ref[...] / jnp.float32(N)
        out_ref[...] = jnp.dot(pooled, wl_ref[...],
                               preferred_element_type=jnp.float32,
                               precision=_PREC) + bl_ref[...]


_stage3 = pl.pallas_call(
    _stage3_body,
    grid=(_GRID,),
    in_specs=[
        pl.BlockSpec((NC, _BLK, D), lambda i: (0, i, 0)),
        pl.BlockSpec((_BLK, 1), lambda i: (i, 0)),
        pl.BlockSpec((1, D), lambda i: (0, 0)),
        pl.BlockSpec((D, 1), lambda i: (0, 0)),
        pl.BlockSpec((1, 1), lambda i: (0, 0)),
    ],
    out_specs=pl.BlockSpec((1, 1), lambda i: (0, 0)),
    out_shape=jax.ShapeDtypeStruct((1, 1), jnp.float32),
    scratch_shapes=[pltpu.VMEM((1, D), jnp.float32)],
)


def kernel(x, edge_index, W1, b1, W2, b2, Wl, bl):
    src = edge_index[0].astype(jnp.int32)
    dst = edge_index[1].astype(jnp.int32)

    ar = jnp.arange(EP - E, dtype=jnp.int32)
    pad_dst = N + ar % (NA - N)
    pad_deg = N + ar % (NAD - N)
    pad_gat = (ar * 41) % N
    dstp = jnp.concatenate([dst, pad_dst]).reshape(NCHP, CH)
    srcd = jnp.concatenate([src, pad_deg]).reshape(NCHP, CH)
    srcg = jnp.concatenate([src, pad_gat]).reshape(NCHP, CH)

    p1 = _mm1(x, W1)
    degp = _sc_degrees(srcd, dstp).reshape(NC, 2, N)
    do0 = degp[0, 0].reshape(N, 1)
    do1 = degp[1, 0].reshape(N, 1)
    di0 = degp[0, 1].reshape(N, 1)
    di1 = degp[1, 1].reshape(N, 1)

    g1, n_out, n_in = _stage1(p1, do0, do1, di0, di1)
    agg1p = _sc_scatter(g1, srcg, dstp)
    g2 = _stage2(agg1p, n_in, n_out, b1.reshape(1, D), W2)
    agg2p = _sc_scatter(g2, srcg, dstp)
    out = _stage3(agg2p, n_in, b2.reshape(1, D), Wl, bl.reshape(1, 1))
    return out
